# baseline (speedup 1.0000x reference)
import functools

import jax
import jax.numpy as jnp
from jax import lax
from jax.experimental import pallas as pl
from jax.experimental.pallas import tpu as pltpu
from jax.experimental.pallas import tpu_sc as plsc

N_NODES = 10000
N_EDGES = 10000
N_INC = 320000
D_IN = 128
DH = 64
NC = 2
NS = 16
R = 10240
RT = R // NS
K = 128
BLK = 32
NBLK = 5
CHUNKS = BLK * NBLK
NCHUNKS = NS * CHUNKS
NI_PAD = NCHUNKS * K
LAG = 2
NBUF = 4


def _mm_body(x_ref, w_ref, o_ref):
    xw = jnp.dot(x_ref[...], w_ref[...],
                 preferred_element_type=jnp.float32)
    o_ref[0, :N_NODES, :] = xw[:, :DH]
    o_ref[1, :N_NODES, :] = xw[:, DH:]
    o_ref[0, N_NODES:, :] = jnp.zeros((R - N_NODES, DH), jnp.float32)
    o_ref[1, N_NODES:, :] = jnp.zeros((R - N_NODES, DH), jnp.float32)


def _matmul_split(x, w):
    return pl.pallas_call(
        _mm_body,
        out_shape=jax.ShapeDtypeStruct((NC, R, DH), jnp.float32),
    )(x, w)


def _epilogue_body(yp_ref, w2_ref, b2_ref, o_ref):
    yp = yp_ref[...]
    y0 = jnp.sum(yp[:NS], axis=0)
    y1 = jnp.sum(yp[NS:], axis=0)
    h = jnp.dot(y0.reshape(1, DH), w2_ref[:DH, :],
                preferred_element_type=jnp.float32)
    h = h + jnp.dot(y1.reshape(1, DH), w2_ref[DH:, :],
                    preferred_element_type=jnp.float32)
    o_ref[...] = h * (1.0 / N_NODES) + b2_ref[...]


def _epilogue(yparts, W2, b2):
    out = pl.pallas_call(
        _epilogue_body,
        out_shape=jax.ShapeDtypeStruct((1, D_IN), jnp.float32),
    )(yparts, W2, b2.reshape(1, D_IN))
    return out.reshape(D_IN)


def _bcast(ref, i):
    v = ref[pl.ds(i, 16)]
    return jnp.broadcast_to(v[0], (16,))


def _sc_body(xw_hbm, vidx_hbm, eidx_hbm, b1_hbm, zrow_hbm, zsc_hbm,
             yout_hbm, oedump_hbm,
             oe_s, degn_s, dege_s, c_s, u_s,
             vblk_v, eblk_v, rows_v, vals_v, ones_v,
             sa_v, sb_v, b1_v, yacc_v,
             sem_a, sem_b, sem_c, sem_d):
    cid = lax.axis_index("c")
    sid = lax.axis_index("s")
    r0 = sid * RT

    z0 = pltpu.async_copy(zrow_hbm, oe_s.at[pl.ds(r0, RT)], sem_a.at[0])
    z2 = pltpu.async_copy(zsc_hbm, degn_s.at[pl.ds(r0, RT)], sem_a.at[2])
    z3 = pltpu.async_copy(zsc_hbm, dege_s.at[pl.ds(r0, RT)], sem_a.at[3])
    z4 = pltpu.async_copy(zsc_hbm, c_s.at[pl.ds(r0, RT)], sem_b.at[0])
    z5 = pltpu.async_copy(zsc_hbm, u_s.at[pl.ds(r0, RT)], sem_b.at[1])
    z6 = pltpu.async_copy(b1_hbm.at[cid], b1_v, sem_b.at[2])
    for z in (z0, z2, z3, z4, z5, z6):
        z.wait()
    for i in range(K // 16):
        ones_v[pl.ds(i * 16, 16)] = jnp.ones((16,), jnp.float32)
    plsc.subcore_barrier()

    def load_blk(b):
        crow = sid * CHUNKS + b * BLK
        pltpu.sync_copy(vidx_hbm.at[pl.ds(crow, BLK)], vblk_v)
        pltpu.sync_copy(eidx_hbm.at[pl.ds(crow, BLK)], eblk_v)

    def phase_a(b, _):
        load_blk(b)
        g = {}
        sc = {}
        cn = {}
        ce = {}

        def consume(k):
            p = k % NBUF
            g[k].wait()
            sc[k] = pltpu.async_copy(rows_v.at[p], oe_s.at[eblk_v.at[k]],
                                     sem_b.at[p], add=True)

        for k in range(BLK):
            p = k % NBUF
            if k >= LAG:
                consume(k - LAG)
            if k >= NBUF:
                sc[k - NBUF].wait()
                cn[k - NBUF].wait()
                ce[k - NBUF].wait()
            cn[k] = pltpu.async_copy(ones_v, degn_s.at[vblk_v.at[k]],
                                     sem_c.at[p], add=True)
            ce[k] = pltpu.async_copy(ones_v, dege_s.at[eblk_v.at[k]],
                                     sem_d.at[p], add=True)
            g[k] = pltpu.async_copy(xw_hbm.at[cid].at[vblk_v.at[k]],
                                    rows_v.at[p], sem_a.at[p])
        for k in range(BLK - LAG, BLK):
            consume(k)
        for k in range(BLK - NBUF, BLK):
            sc[k].wait()
            cn[k].wait()
            ce[k].wait()
        return 0
    lax.fori_loop(0, NBLK, phase_a, 0)
    plsc.subcore_barrier()

    pltpu.sync_copy(degn_s.at[pl.ds(r0, RT)], sa_v.at[pl.ds(0, RT)])
    pltpu.sync_copy(dege_s.at[pl.ds(r0, RT)], sb_v.at[pl.ds(0, RT)])
    for i in range(RT // 16):
        s = pl.ds(i * 16, 16)
        x = sa_v[s]
        sa_v[s] = jnp.where(x > 0.0, 1.0 / jnp.where(x > 0.0, x, 1.0), 0.0)
        y = sb_v[s]
        sb_v[s] = jnp.where(y > 0.0, 1.0 / jnp.where(y > 0.0, y, 1.0), 0.0)
    pltpu.sync_copy(sa_v.at[pl.ds(0, RT)], degn_s.at[pl.ds(r0, RT)])
    pltpu.sync_copy(sb_v.at[pl.ds(0, RT)], dege_s.at[pl.ds(r0, RT)])

    zb = {}
    for blk in range(RT // K):
        pltpu.sync_copy(oe_s.at[pl.ds(r0 + blk * K, K)], rows_v.at[0])

        def scale_row(r, _, blk=blk):
            bb = _bcast(sb_v, blk * K + r)
            for c4 in range(DH // 16):
                s = pl.ds(c4 * 16, 16)
                rows_v[0, r, s] = rows_v[0, r, s] * bb
            return 0
        lax.fori_loop(0, K, scale_row, 0)
        pltpu.sync_copy(rows_v.at[0],
                        oedump_hbm.at[cid].at[pl.ds(r0 + blk * K, K)])
        if blk >= NBUF:
            zb[blk - NBUF].wait()
        zb[blk] = pltpu.async_copy(zrow_hbm.at[pl.ds(0, K)],
                                   oe_s.at[pl.ds(r0 + blk * K, K)],
                                   sem_c.at[blk % NBUF])
    for blk in range(RT // K - NBUF, RT // K):
        if blk >= 0:
            zb[blk].wait()
    plsc.subcore_barrier()

    def phase_c(b, _):
        load_blk(b)
        gv = {}
        sc = {}

        def consume(k):
            p = k % NBUF
            gv[k].wait()
            sc[k] = pltpu.async_copy(vals_v.at[p], c_s.at[eblk_v.at[k]],
                                     sem_b.at[p], add=True)

        for k in range(BLK):
            p = k % NBUF
            if k >= LAG:
                consume(k - LAG)
            if k >= NBUF:
                sc[k - NBUF].wait()
            gv[k] = pltpu.async_copy(degn_s.at[vblk_v.at[k]], vals_v.at[p],
                                     sem_a.at[p])
        for k in range(BLK - LAG, BLK):
            consume(k)
        for k in range(BLK - NBUF, BLK):
            sc[k].wait()
        return 0
    lax.fori_loop(0, NBLK, phase_c, 0)
    plsc.subcore_barrier()

    pltpu.sync_copy(c_s.at[pl.ds(r0, RT)], sa_v.at[pl.ds(0, RT)])
    for i in range(RT // 16):
        s = pl.ds(i * 16, 16)
        sa_v[s] = sa_v[s] * sb_v[s]
    pltpu.sync_copy(sa_v.at[pl.ds(0, RT)], c_s.at[pl.ds(r0, RT)])
    plsc.subcore_barrier()

    def phase_e(b, _):
        load_blk(b)
        g = {}
        gv = {}
        sn = {}
        su = {}

        def consume(k):
            p = k % NBUF
            gv[k].wait()
            su[k] = pltpu.async_copy(vals_v.at[p], u_s.at[vblk_v.at[k]],
                                     sem_c.at[p], add=True)
            g[k].wait()
            sn[k] = pltpu.async_copy(rows_v.at[p], oe_s.at[vblk_v.at[k]],
                                     sem_b.at[p], add=True)

        for k in range(BLK):
            p = k % NBUF
            if k >= LAG:
                consume(k - LAG)
            if k >= NBUF:
                sn[k - NBUF].wait()
                su[k - NBUF].wait()
            gv[k] = pltpu.async_copy(c_s.at[eblk_v.at[k]], vals_v.at[p],
                                     sem_a.at[p])
            g[k] = pltpu.async_copy(oedump_hbm.at[cid].at[eblk_v.at[k]],
                                    rows_v.at[p], sem_d.at[p])
        for k in range(BLK - LAG, BLK):
            consume(k)
        for k in range(BLK - NBUF, BLK):
            sn[k].wait()
            su[k].wait()
        return 0
    lax.fori_loop(0, NBLK, phase_e, 0)
    plsc.subcore_barrier()

    pltpu.sync_copy(degn_s.at[pl.ds(r0, RT)], sa_v.at[pl.ds(0, RT)])
    pltpu.sync_copy(u_s.at[pl.ds(r0, RT)], sb_v.at[pl.ds(0, RT)])
    for c4 in range(DH // 16):
        yacc_v[pl.ds(c4 * 16, 16)] = jnp.zeros((16,), jnp.float32)

    nvalid = jnp.minimum(RT, jnp.maximum(0, N_NODES - r0))

    for blk in range(RT // K):
        pltpu.sync_copy(oe_s.at[pl.ds(r0 + blk * K, K)], rows_v.at[0])
        nv_blk = jnp.clip(nvalid - blk * K, 0, K)

        def yrow(r, _, blk=blk):
            d = _bcast(sa_v, blk * K + r)
            u = _bcast(sb_v, blk * K + r)
            for c4 in range(DH // 16):
                s = pl.ds(c4 * 16, 16)
                z = jnp.maximum(d * rows_v[0, r, s] + b1_v[s], 0.0)
                yacc_v[s] = yacc_v[s] + u * z
            return 0
        lax.fori_loop(0, nv_blk, yrow, 0)
    pltpu.sync_copy(yacc_v, yout_hbm.at[cid * NS + sid])


_sc_kernel = functools.partial(
    pl.kernel,
    _sc_body,
    out_type=(jax.ShapeDtypeStruct((NC * NS, DH), jnp.float32),
              jax.ShapeDtypeStruct((NC, R, DH), jnp.float32)),
    mesh=plsc.VectorSubcoreMesh(core_axis_name="c", subcore_axis_name="s"),
    compiler_params=pltpu.CompilerParams(use_tc_tiling_on_sc=False),
    scratch_types=[
        pltpu.VMEM_SHARED((R, DH), jnp.float32),
        pltpu.VMEM_SHARED((R,), jnp.float32),
        pltpu.VMEM_SHARED((R,), jnp.float32),
        pltpu.VMEM_SHARED((R,), jnp.float32),
        pltpu.VMEM_SHARED((R,), jnp.float32),
        pltpu.VMEM((BLK, K), jnp.int32),
        pltpu.VMEM((BLK, K), jnp.int32),
        pltpu.VMEM((NBUF, K, DH), jnp.float32),
        pltpu.VMEM((NBUF, K), jnp.float32),
        pltpu.VMEM((K,), jnp.float32),
        pltpu.VMEM((RT + 16,), jnp.float32),
        pltpu.VMEM((RT + 16,), jnp.float32),
        pltpu.VMEM((DH,), jnp.float32),
        pltpu.VMEM((DH,), jnp.float32),
        pltpu.SemaphoreType.DMA((NBUF,)),
        pltpu.SemaphoreType.DMA((NBUF,)),
        pltpu.SemaphoreType.DMA((NBUF,)),
        pltpu.SemaphoreType.DMA((NBUF,)),
    ],
)()


@jax.jit
def kernel(x_phy, hyperedge_index, W1, b1, W2, b2):
    xw3 = _matmul_split(x_phy, W1)

    npad = NI_PAD - N_INC
    pad_ids = (N_NODES + (jnp.arange(npad, dtype=jnp.int32) % (R - N_NODES)))
    vidx = jnp.concatenate([hyperedge_index[0], pad_ids]).reshape(NCHUNKS, K)
    eidx = jnp.concatenate([hyperedge_index[1], pad_ids]).reshape(NCHUNKS, K)

    zrow = jnp.zeros((RT, DH), jnp.float32)
    zsc = jnp.zeros((RT,), jnp.float32)
    b1r = b1.reshape(NC, DH)

    yparts, _unused_oe = _sc_kernel(xw3, vidx, eidx, b1r, zrow, zsc)
    return _epilogue(yparts, W2, b2)


# speedup vs baseline: 1.0618x; 1.0618x over previous
import functools

import jax
import jax.numpy as jnp
from jax import lax
from jax.experimental import pallas as pl
from jax.experimental.pallas import tpu as pltpu
from jax.experimental.pallas import tpu_sc as plsc

N_NODES = 10000
N_EDGES = 10000
N_INC = 320000
D_IN = 128
DH = 64
NC = 2
NS = 16
R = 10240
RT = R // NS
K = 128
BLK = 32
NBLK = 5
CHUNKS = BLK * NBLK
NCHUNKS = NS * CHUNKS
NI_PAD = NCHUNKS * K
LAG = 3
NBUF = 6


def _mm_body(x_ref, w_ref, o_ref):
    xw = jnp.dot(x_ref[...], w_ref[...],
                 preferred_element_type=jnp.float32)
    o_ref[0, :N_NODES, :] = xw[:, :DH]
    o_ref[1, :N_NODES, :] = xw[:, DH:]
    o_ref[0, N_NODES:, :] = jnp.zeros((R - N_NODES, DH), jnp.float32)
    o_ref[1, N_NODES:, :] = jnp.zeros((R - N_NODES, DH), jnp.float32)


def _matmul_split(x, w):
    return pl.pallas_call(
        _mm_body,
        out_shape=jax.ShapeDtypeStruct((NC, R, DH), jnp.float32),
    )(x, w)


def _epilogue_body(yp_ref, w2_ref, b2_ref, o_ref):
    yp = yp_ref[...]
    y0 = jnp.sum(yp[:NS], axis=0)
    y1 = jnp.sum(yp[NS:], axis=0)
    h = jnp.dot(y0.reshape(1, DH), w2_ref[:DH, :],
                preferred_element_type=jnp.float32)
    h = h + jnp.dot(y1.reshape(1, DH), w2_ref[DH:, :],
                    preferred_element_type=jnp.float32)
    o_ref[...] = h * (1.0 / N_NODES) + b2_ref[...]


def _epilogue(yparts, W2, b2):
    out = pl.pallas_call(
        _epilogue_body,
        out_shape=jax.ShapeDtypeStruct((1, D_IN), jnp.float32),
    )(yparts, W2, b2.reshape(1, D_IN))
    return out.reshape(D_IN)


def _bcast(ref, i):
    v = ref[pl.ds(i, 16)]
    return jnp.broadcast_to(v[0], (16,))


def _sc_body(xw_hbm, vidx_hbm, eidx_hbm, b1_hbm, zrow_hbm, zsc_hbm,
             yout_hbm, oedump_hbm,
             oe_s, degn_s, dege_s, c_s, u_s,
             vblk_v, eblk_v, rows_v, vals_v, ones_v,
             sa_v, sb_v, b1_v, yacc_v,
             sem_a, sem_b, sem_c, sem_d):
    cid = lax.axis_index("c")
    sid = lax.axis_index("s")
    r0 = sid * RT

    z0 = pltpu.async_copy(zrow_hbm, oe_s.at[pl.ds(r0, RT)], sem_a.at[0])
    z2 = pltpu.async_copy(zsc_hbm, degn_s.at[pl.ds(r0, RT)], sem_a.at[2])
    z3 = pltpu.async_copy(zsc_hbm, dege_s.at[pl.ds(r0, RT)], sem_a.at[3])
    z4 = pltpu.async_copy(zsc_hbm, c_s.at[pl.ds(r0, RT)], sem_b.at[0])
    z5 = pltpu.async_copy(zsc_hbm, u_s.at[pl.ds(r0, RT)], sem_b.at[1])
    z6 = pltpu.async_copy(b1_hbm.at[cid], b1_v, sem_b.at[2])
    for z in (z0, z2, z3, z4, z5, z6):
        z.wait()
    for i in range(K // 16):
        ones_v[pl.ds(i * 16, 16)] = jnp.ones((16,), jnp.float32)
    plsc.subcore_barrier()

    def load_blk(b):
        crow = sid * CHUNKS + b * BLK
        pltpu.sync_copy(vidx_hbm.at[pl.ds(crow, BLK)], vblk_v)
        pltpu.sync_copy(eidx_hbm.at[pl.ds(crow, BLK)], eblk_v)

    def phase_a(b, _):
        load_blk(b)
        g = {}
        sc = {}
        cn = {}
        ce = {}

        def consume(k):
            p = k % NBUF
            g[k].wait()
            sc[k] = pltpu.async_copy(rows_v.at[p], oe_s.at[eblk_v.at[k]],
                                     sem_b.at[p], add=True)

        for k in range(BLK):
            p = k % NBUF
            if k >= LAG:
                consume(k - LAG)
            if k >= NBUF:
                sc[k - NBUF].wait()
                cn[k - NBUF].wait()
                ce[k - NBUF].wait()
            cn[k] = pltpu.async_copy(ones_v, degn_s.at[vblk_v.at[k]],
                                     sem_c.at[p], add=True)
            ce[k] = pltpu.async_copy(ones_v, dege_s.at[eblk_v.at[k]],
                                     sem_d.at[p], add=True)
            g[k] = pltpu.async_copy(xw_hbm.at[cid].at[vblk_v.at[k]],
                                    rows_v.at[p], sem_a.at[p])
        for k in range(BLK - LAG, BLK):
            consume(k)
        for k in range(BLK - NBUF, BLK):
            sc[k].wait()
            cn[k].wait()
            ce[k].wait()
        return 0
    lax.fori_loop(0, NBLK, phase_a, 0)
    plsc.subcore_barrier()

    pltpu.sync_copy(degn_s.at[pl.ds(r0, RT)], sa_v.at[pl.ds(0, RT)])
    pltpu.sync_copy(dege_s.at[pl.ds(r0, RT)], sb_v.at[pl.ds(0, RT)])
    for i in range(RT // 16):
        s = pl.ds(i * 16, 16)
        x = sa_v[s]
        sa_v[s] = jnp.where(x > 0.0, 1.0 / jnp.where(x > 0.0, x, 1.0), 0.0)
        y = sb_v[s]
        sb_v[s] = jnp.where(y > 0.0, 1.0 / jnp.where(y > 0.0, y, 1.0), 0.0)
    pltpu.sync_copy(sa_v.at[pl.ds(0, RT)], degn_s.at[pl.ds(r0, RT)])
    pltpu.sync_copy(sb_v.at[pl.ds(0, RT)], dege_s.at[pl.ds(r0, RT)])

    zb = {}
    for blk in range(RT // K):
        pltpu.sync_copy(oe_s.at[pl.ds(r0 + blk * K, K)], rows_v.at[0])

        def scale_row(r, _, blk=blk):
            bb = _bcast(sb_v, blk * K + r)
            for c4 in range(DH // 16):
                s = pl.ds(c4 * 16, 16)
                rows_v[0, r, s] = rows_v[0, r, s] * bb
            return 0
        lax.fori_loop(0, K, scale_row, 0)
        pltpu.sync_copy(rows_v.at[0],
                        oedump_hbm.at[cid].at[pl.ds(r0 + blk * K, K)])
        if blk >= NBUF:
            zb[blk - NBUF].wait()
        zb[blk] = pltpu.async_copy(zrow_hbm.at[pl.ds(0, K)],
                                   oe_s.at[pl.ds(r0 + blk * K, K)],
                                   sem_c.at[blk % NBUF])
    for blk in range(RT // K - NBUF, RT // K):
        if blk >= 0:
            zb[blk].wait()
    plsc.subcore_barrier()

    def phase_c(b, _):
        load_blk(b)
        gv = {}
        sc = {}

        def consume(k):
            p = k % NBUF
            gv[k].wait()
            sc[k] = pltpu.async_copy(vals_v.at[p], c_s.at[eblk_v.at[k]],
                                     sem_b.at[p], add=True)

        for k in range(BLK):
            p = k % NBUF
            if k >= LAG:
                consume(k - LAG)
            if k >= NBUF:
                sc[k - NBUF].wait()
            gv[k] = pltpu.async_copy(degn_s.at[vblk_v.at[k]], vals_v.at[p],
                                     sem_a.at[p])
        for k in range(BLK - LAG, BLK):
            consume(k)
        for k in range(BLK - NBUF, BLK):
            sc[k].wait()
        return 0
    lax.fori_loop(0, NBLK, phase_c, 0)
    plsc.subcore_barrier()

    pltpu.sync_copy(c_s.at[pl.ds(r0, RT)], sa_v.at[pl.ds(0, RT)])
    for i in range(RT // 16):
        s = pl.ds(i * 16, 16)
        sa_v[s] = sa_v[s] * sb_v[s]
    pltpu.sync_copy(sa_v.at[pl.ds(0, RT)], c_s.at[pl.ds(r0, RT)])
    plsc.subcore_barrier()

    def phase_e(b, _):
        load_blk(b)
        g = {}
        gv = {}
        sn = {}
        su = {}

        def consume(k):
            p = k % NBUF
            gv[k].wait()
            su[k] = pltpu.async_copy(vals_v.at[p], u_s.at[vblk_v.at[k]],
                                     sem_c.at[p], add=True)
            g[k].wait()
            sn[k] = pltpu.async_copy(rows_v.at[p], oe_s.at[vblk_v.at[k]],
                                     sem_b.at[p], add=True)

        for k in range(BLK):
            p = k % NBUF
            if k >= LAG:
                consume(k - LAG)
            if k >= NBUF:
                sn[k - NBUF].wait()
                su[k - NBUF].wait()
            gv[k] = pltpu.async_copy(c_s.at[eblk_v.at[k]], vals_v.at[p],
                                     sem_a.at[p])
            g[k] = pltpu.async_copy(oedump_hbm.at[cid].at[eblk_v.at[k]],
                                    rows_v.at[p], sem_d.at[p])
        for k in range(BLK - LAG, BLK):
            consume(k)
        for k in range(BLK - NBUF, BLK):
            sn[k].wait()
            su[k].wait()
        return 0
    lax.fori_loop(0, NBLK, phase_e, 0)
    plsc.subcore_barrier()

    pltpu.sync_copy(degn_s.at[pl.ds(r0, RT)], sa_v.at[pl.ds(0, RT)])
    pltpu.sync_copy(u_s.at[pl.ds(r0, RT)], sb_v.at[pl.ds(0, RT)])
    for c4 in range(DH // 16):
        yacc_v[pl.ds(c4 * 16, 16)] = jnp.zeros((16,), jnp.float32)

    nvalid = jnp.minimum(RT, jnp.maximum(0, N_NODES - r0))

    for blk in range(RT // K):
        pltpu.sync_copy(oe_s.at[pl.ds(r0 + blk * K, K)], rows_v.at[0])
        nv_blk = jnp.clip(nvalid - blk * K, 0, K)

        def yrow(r, _, blk=blk):
            d = _bcast(sa_v, blk * K + r)
            u = _bcast(sb_v, blk * K + r)
            for c4 in range(DH // 16):
                s = pl.ds(c4 * 16, 16)
                z = jnp.maximum(d * rows_v[0, r, s] + b1_v[s], 0.0)
                yacc_v[s] = yacc_v[s] + u * z
            return 0
        lax.fori_loop(0, nv_blk, yrow, 0)
    pltpu.sync_copy(yacc_v, yout_hbm.at[cid * NS + sid])


_sc_kernel = functools.partial(
    pl.kernel,
    _sc_body,
    out_type=(jax.ShapeDtypeStruct((NC * NS, DH), jnp.float32),
              jax.ShapeDtypeStruct((NC, R, DH), jnp.float32)),
    mesh=plsc.VectorSubcoreMesh(core_axis_name="c", subcore_axis_name="s"),
    compiler_params=pltpu.CompilerParams(use_tc_tiling_on_sc=False),
    scratch_types=[
        pltpu.VMEM_SHARED((R, DH), jnp.float32),
        pltpu.VMEM_SHARED((R,), jnp.float32),
        pltpu.VMEM_SHARED((R,), jnp.float32),
        pltpu.VMEM_SHARED((R,), jnp.float32),
        pltpu.VMEM_SHARED((R,), jnp.float32),
        pltpu.VMEM((BLK, K), jnp.int32),
        pltpu.VMEM((BLK, K), jnp.int32),
        pltpu.VMEM((NBUF, K, DH), jnp.float32),
        pltpu.VMEM((NBUF, K), jnp.float32),
        pltpu.VMEM((K,), jnp.float32),
        pltpu.VMEM((RT + 16,), jnp.float32),
        pltpu.VMEM((RT + 16,), jnp.float32),
        pltpu.VMEM((DH,), jnp.float32),
        pltpu.VMEM((DH,), jnp.float32),
        pltpu.SemaphoreType.DMA((NBUF,)),
        pltpu.SemaphoreType.DMA((NBUF,)),
        pltpu.SemaphoreType.DMA((NBUF,)),
        pltpu.SemaphoreType.DMA((NBUF,)),
    ],
)()


@jax.jit
def kernel(x_phy, hyperedge_index, W1, b1, W2, b2):
    xw3 = _matmul_split(x_phy, W1)

    npad = NI_PAD - N_INC
    pad_ids = (N_NODES + (jnp.arange(npad, dtype=jnp.int32) % (R - N_NODES)))
    vidx = jnp.concatenate([hyperedge_index[0], pad_ids]).reshape(NCHUNKS, K)
    eidx = jnp.concatenate([hyperedge_index[1], pad_ids]).reshape(NCHUNKS, K)

    zrow = jnp.zeros((RT, DH), jnp.float32)
    zsc = jnp.zeros((RT,), jnp.float32)
    b1r = b1.reshape(NC, DH)

    yparts, _unused_oe = _sc_kernel(xw3, vidx, eidx, b1r, zrow, zsc)
    return _epilogue(yparts, W2, b2)


# speedup vs baseline: 1.0623x; 1.0005x over previous
import functools

import jax
import jax.numpy as jnp
from jax import lax
from jax.experimental import pallas as pl
from jax.experimental.pallas import tpu as pltpu
from jax.experimental.pallas import tpu_sc as plsc

N_NODES = 10000
N_EDGES = 10000
N_INC = 320000
D_IN = 128
DH = 64
NC = 2
NS = 16
R = 10240
RT = R // NS
K = 128
BLK = 32
NBLK = 5
CHUNKS = BLK * NBLK
NCHUNKS = NS * CHUNKS
NI_PAD = NCHUNKS * K
LAG = 3
NBUF = 7


def _mm_body(x_ref, w_ref, o_ref):
    xw = jnp.dot(x_ref[...], w_ref[...],
                 preferred_element_type=jnp.float32)
    o_ref[0, :N_NODES, :] = xw[:, :DH]
    o_ref[1, :N_NODES, :] = xw[:, DH:]
    o_ref[0, N_NODES:, :] = jnp.zeros((R - N_NODES, DH), jnp.float32)
    o_ref[1, N_NODES:, :] = jnp.zeros((R - N_NODES, DH), jnp.float32)


def _matmul_split(x, w):
    return pl.pallas_call(
        _mm_body,
        out_shape=jax.ShapeDtypeStruct((NC, R, DH), jnp.float32),
    )(x, w)


def _epilogue_body(yp_ref, w2_ref, b2_ref, o_ref):
    yp = yp_ref[...]
    y0 = jnp.sum(yp[:NS], axis=0)
    y1 = jnp.sum(yp[NS:], axis=0)
    h = jnp.dot(y0.reshape(1, DH), w2_ref[:DH, :],
                preferred_element_type=jnp.float32)
    h = h + jnp.dot(y1.reshape(1, DH), w2_ref[DH:, :],
                    preferred_element_type=jnp.float32)
    o_ref[...] = h * (1.0 / N_NODES) + b2_ref[...]


def _epilogue(yparts, W2, b2):
    out = pl.pallas_call(
        _epilogue_body,
        out_shape=jax.ShapeDtypeStruct((1, D_IN), jnp.float32),
    )(yparts, W2, b2.reshape(1, D_IN))
    return out.reshape(D_IN)


def _bcast(ref, i):
    v = ref[pl.ds(i, 16)]
    return jnp.broadcast_to(v[0], (16,))


def _sc_body(xw_hbm, vidx_hbm, eidx_hbm, b1_hbm, zrow_hbm, zsc_hbm,
             yout_hbm, oedump_hbm,
             oe_s, degn_s, dege_s, c_s, u_s,
             vblk_v, eblk_v, rows_v, vals_v, ones_v,
             sa_v, sb_v, b1_v, yacc_v,
             sem_a, sem_b, sem_c, sem_d):
    cid = lax.axis_index("c")
    sid = lax.axis_index("s")
    r0 = sid * RT

    z0 = pltpu.async_copy(zrow_hbm, oe_s.at[pl.ds(r0, RT)], sem_a.at[0])
    z2 = pltpu.async_copy(zsc_hbm, degn_s.at[pl.ds(r0, RT)], sem_a.at[2])
    z3 = pltpu.async_copy(zsc_hbm, dege_s.at[pl.ds(r0, RT)], sem_a.at[3])
    z4 = pltpu.async_copy(zsc_hbm, c_s.at[pl.ds(r0, RT)], sem_b.at[0])
    z5 = pltpu.async_copy(zsc_hbm, u_s.at[pl.ds(r0, RT)], sem_b.at[1])
    z6 = pltpu.async_copy(b1_hbm.at[cid], b1_v, sem_b.at[2])
    for z in (z0, z2, z3, z4, z5, z6):
        z.wait()
    for i in range(K // 16):
        ones_v[pl.ds(i * 16, 16)] = jnp.ones((16,), jnp.float32)
    plsc.subcore_barrier()

    def load_blk(b):
        crow = sid * CHUNKS + b * BLK
        pltpu.sync_copy(vidx_hbm.at[pl.ds(crow, BLK)], vblk_v)
        pltpu.sync_copy(eidx_hbm.at[pl.ds(crow, BLK)], eblk_v)

    def phase_a(b, _):
        load_blk(b)
        g = {}
        sc = {}
        cn = {}
        ce = {}

        def consume(k):
            p = k % NBUF
            g[k].wait()
            sc[k] = pltpu.async_copy(rows_v.at[p], oe_s.at[eblk_v.at[k]],
                                     sem_b.at[p], add=True)

        for k in range(BLK):
            p = k % NBUF
            if k >= LAG:
                consume(k - LAG)
            if k >= NBUF:
                sc[k - NBUF].wait()
                cn[k - NBUF].wait()
                ce[k - NBUF].wait()
            cn[k] = pltpu.async_copy(ones_v, degn_s.at[vblk_v.at[k]],
                                     sem_c.at[p], add=True)
            ce[k] = pltpu.async_copy(ones_v, dege_s.at[eblk_v.at[k]],
                                     sem_d.at[p], add=True)
            g[k] = pltpu.async_copy(xw_hbm.at[cid].at[vblk_v.at[k]],
                                    rows_v.at[p], sem_a.at[p])
        for k in range(BLK - LAG, BLK):
            consume(k)
        for k in range(BLK - NBUF, BLK):
            sc[k].wait()
            cn[k].wait()
            ce[k].wait()
        return 0
    lax.fori_loop(0, NBLK, phase_a, 0)
    plsc.subcore_barrier()

    pltpu.sync_copy(degn_s.at[pl.ds(r0, RT)], sa_v.at[pl.ds(0, RT)])
    pltpu.sync_copy(dege_s.at[pl.ds(r0, RT)], sb_v.at[pl.ds(0, RT)])
    for i in range(RT // 16):
        s = pl.ds(i * 16, 16)
        x = sa_v[s]
        sa_v[s] = jnp.where(x > 0.0, 1.0 / jnp.where(x > 0.0, x, 1.0), 0.0)
        y = sb_v[s]
        sb_v[s] = jnp.where(y > 0.0, 1.0 / jnp.where(y > 0.0, y, 1.0), 0.0)
    pltpu.sync_copy(sa_v.at[pl.ds(0, RT)], degn_s.at[pl.ds(r0, RT)])
    pltpu.sync_copy(sb_v.at[pl.ds(0, RT)], dege_s.at[pl.ds(r0, RT)])

    zb = {}
    for blk in range(RT // K):
        pltpu.sync_copy(oe_s.at[pl.ds(r0 + blk * K, K)], rows_v.at[0])

        def scale_row(r, _, blk=blk):
            bb = _bcast(sb_v, blk * K + r)
            for c4 in range(DH // 16):
                s = pl.ds(c4 * 16, 16)
                rows_v[0, r, s] = rows_v[0, r, s] * bb
            return 0
        lax.fori_loop(0, K, scale_row, 0)
        pltpu.sync_copy(rows_v.at[0],
                        oedump_hbm.at[cid].at[pl.ds(r0 + blk * K, K)])
        if blk >= NBUF:
            zb[blk - NBUF].wait()
        zb[blk] = pltpu.async_copy(zrow_hbm.at[pl.ds(0, K)],
                                   oe_s.at[pl.ds(r0 + blk * K, K)],
                                   sem_c.at[blk % NBUF])
    for blk in range(RT // K - NBUF, RT // K):
        if blk >= 0:
            zb[blk].wait()
    plsc.subcore_barrier()

    def phase_c(b, _):
        load_blk(b)
        gv = {}
        sc = {}

        def consume(k):
            p = k % NBUF
            gv[k].wait()
            sc[k] = pltpu.async_copy(vals_v.at[p], c_s.at[eblk_v.at[k]],
                                     sem_b.at[p], add=True)

        for k in range(BLK):
            p = k % NBUF
            if k >= LAG:
                consume(k - LAG)
            if k >= NBUF:
                sc[k - NBUF].wait()
            gv[k] = pltpu.async_copy(degn_s.at[vblk_v.at[k]], vals_v.at[p],
                                     sem_a.at[p])
        for k in range(BLK - LAG, BLK):
            consume(k)
        for k in range(BLK - NBUF, BLK):
            sc[k].wait()
        return 0
    lax.fori_loop(0, NBLK, phase_c, 0)
    plsc.subcore_barrier()

    pltpu.sync_copy(c_s.at[pl.ds(r0, RT)], sa_v.at[pl.ds(0, RT)])
    for i in range(RT // 16):
        s = pl.ds(i * 16, 16)
        sa_v[s] = sa_v[s] * sb_v[s]
    pltpu.sync_copy(sa_v.at[pl.ds(0, RT)], c_s.at[pl.ds(r0, RT)])
    plsc.subcore_barrier()

    def phase_e(b, _):
        load_blk(b)
        g = {}
        gv = {}
        sn = {}
        su = {}

        def consume(k):
            p = k % NBUF
            gv[k].wait()
            su[k] = pltpu.async_copy(vals_v.at[p], u_s.at[vblk_v.at[k]],
                                     sem_c.at[p], add=True)
            g[k].wait()
            sn[k] = pltpu.async_copy(rows_v.at[p], oe_s.at[vblk_v.at[k]],
                                     sem_b.at[p], add=True)

        for k in range(BLK):
            p = k % NBUF
            if k >= LAG:
                consume(k - LAG)
            if k >= NBUF:
                sn[k - NBUF].wait()
                su[k - NBUF].wait()
            gv[k] = pltpu.async_copy(c_s.at[eblk_v.at[k]], vals_v.at[p],
                                     sem_a.at[p])
            g[k] = pltpu.async_copy(oedump_hbm.at[cid].at[eblk_v.at[k]],
                                    rows_v.at[p], sem_d.at[p])
        for k in range(BLK - LAG, BLK):
            consume(k)
        for k in range(BLK - NBUF, BLK):
            sn[k].wait()
            su[k].wait()
        return 0
    lax.fori_loop(0, NBLK, phase_e, 0)
    plsc.subcore_barrier()

    pltpu.sync_copy(degn_s.at[pl.ds(r0, RT)], sa_v.at[pl.ds(0, RT)])
    pltpu.sync_copy(u_s.at[pl.ds(r0, RT)], sb_v.at[pl.ds(0, RT)])
    for c4 in range(DH // 16):
        yacc_v[pl.ds(c4 * 16, 16)] = jnp.zeros((16,), jnp.float32)

    nvalid = jnp.minimum(RT, jnp.maximum(0, N_NODES - r0))

    for blk in range(RT // K):
        pltpu.sync_copy(oe_s.at[pl.ds(r0 + blk * K, K)], rows_v.at[0])
        nv_blk = jnp.clip(nvalid - blk * K, 0, K)

        def yrow(r, _, blk=blk):
            d = _bcast(sa_v, blk * K + r)
            u = _bcast(sb_v, blk * K + r)
            for c4 in range(DH // 16):
                s = pl.ds(c4 * 16, 16)
                z = jnp.maximum(d * rows_v[0, r, s] + b1_v[s], 0.0)
                yacc_v[s] = yacc_v[s] + u * z
            return 0
        lax.fori_loop(0, nv_blk, yrow, 0)
    pltpu.sync_copy(yacc_v, yout_hbm.at[cid * NS + sid])


_sc_kernel = functools.partial(
    pl.kernel,
    _sc_body,
    out_type=(jax.ShapeDtypeStruct((NC * NS, DH), jnp.float32),
              jax.ShapeDtypeStruct((NC, R, DH), jnp.float32)),
    mesh=plsc.VectorSubcoreMesh(core_axis_name="c", subcore_axis_name="s"),
    compiler_params=pltpu.CompilerParams(use_tc_tiling_on_sc=False),
    scratch_types=[
        pltpu.VMEM_SHARED((R, DH), jnp.float32),
        pltpu.VMEM_SHARED((R,), jnp.float32),
        pltpu.VMEM_SHARED((R,), jnp.float32),
        pltpu.VMEM_SHARED((R,), jnp.float32),
        pltpu.VMEM_SHARED((R,), jnp.float32),
        pltpu.VMEM((BLK, K), jnp.int32),
        pltpu.VMEM((BLK, K), jnp.int32),
        pltpu.VMEM((NBUF, K, DH), jnp.float32),
        pltpu.VMEM((NBUF, K), jnp.float32),
        pltpu.VMEM((K,), jnp.float32),
        pltpu.VMEM((RT + 16,), jnp.float32),
        pltpu.VMEM((RT + 16,), jnp.float32),
        pltpu.VMEM((DH,), jnp.float32),
        pltpu.VMEM((DH,), jnp.float32),
        pltpu.SemaphoreType.DMA((NBUF,)),
        pltpu.SemaphoreType.DMA((NBUF,)),
        pltpu.SemaphoreType.DMA((NBUF,)),
        pltpu.SemaphoreType.DMA((NBUF,)),
    ],
)()


@jax.jit
def kernel(x_phy, hyperedge_index, W1, b1, W2, b2):
    xw3 = _matmul_split(x_phy, W1)

    npad = NI_PAD - N_INC
    pad_ids = (N_NODES + (jnp.arange(npad, dtype=jnp.int32) % (R - N_NODES)))
    vidx = jnp.concatenate([hyperedge_index[0], pad_ids]).reshape(NCHUNKS, K)
    eidx = jnp.concatenate([hyperedge_index[1], pad_ids]).reshape(NCHUNKS, K)

    zrow = jnp.zeros((RT, DH), jnp.float32)
    zsc = jnp.zeros((RT,), jnp.float32)
    b1r = b1.reshape(NC, DH)

    yparts, _unused_oe = _sc_kernel(xw3, vidx, eidx, b1r, zrow, zsc)
    return _epilogue(yparts, W2, b2)
